# baseline (speedup 1.0000x reference)
import dataclasses

import jax
import jax.numpy as jnp
from jax import lax
from jax.experimental import pallas as pl
from jax.experimental.pallas import tpu as pltpu
from jax.experimental.pallas import tpu_sc as plsc

N = 10000
D = 128
E = 320000
NACC = 10240
NROWS = 10240

NP = 5
FP = 31
DP = 32

NC = 2
NS = 16
NW = NC * NS
SUBCH = 128
CH = 512
KSUB = CH // SUBCH
EW = 10240
TOT = EW * NW
NCHUNK = EW // CH
ZROWS = NROWS // NS

R_PRE = 1000


def _pre_body(x_ref, wt_ref, b_ref, w1_ref, b1_ref, w2_ref, b2_ref,
              hp_ref, a1_ref, a2_ref, c_ref, mx_ref):
    i = pl.program_id(0)
    h = jnp.dot(x_ref[...], wt_ref[...], preferred_element_type=jnp.float32)
    h = h + b_ref[...]
    jc = lax.broadcasted_iota(jnp.int32, (D, NP * DP), 1)
    ir = lax.broadcasted_iota(jnp.int32, (D, NP * DP), 0)
    pp = jc // DP
    rr = jc % DP
    num_mask = (jc < 4 * DP) & (rr < FP) & (ir == pp * FP + rr)
    last_mask = (jc >= 4 * DP) & (jc < 4 * DP + 4) & (ir == 4 * FP + jc - 4 * DP)
    sel = jnp.where(num_mask | last_mask, 1.0, 0.0).astype(jnp.float32)
    hpv = lax.dot_general(h, sel, (((1,), (0,)), ((), ())),
                          precision=lax.Precision.HIGHEST,
                          preferred_element_type=jnp.float32)
    jc2 = lax.broadcasted_iota(jnp.int32, (R_PRE, NP * DP), 1)
    onescols = jnp.where(jc2 % DP == DP - 1, 1.0, 0.0).astype(jnp.float32)
    hp_ref[...] = hpv + onescols
    a1 = jnp.sum(h * w1_ref[...], axis=1, keepdims=True) + b1_ref[0]
    a2 = jnp.sum(h * w2_ref[...], axis=1, keepdims=True) + b2_ref[0]
    a1_ref[...] = a1
    a2_ref[...] = a2
    m1 = jnp.max(a1)
    m2 = jnp.max(a2)

    @pl.when(i == 0)
    def _():
        mx_ref[0] = m1
        mx_ref[1] = m2

    @pl.when(i > 0)
    def _():
        mx_ref[0] = jnp.maximum(mx_ref[0], m1)
        mx_ref[1] = jnp.maximum(mx_ref[1], m2)

    c_ref[...] = jnp.full((16,), jnp.maximum(mx_ref[0] + mx_ref[1], 0.0),
                          jnp.float32)


def _tc_pre(x, wt, b2d, w1, b1, w2, b2):
    return pl.pallas_call(
        _pre_body,
        grid=(N // R_PRE,),
        in_specs=[
            pl.BlockSpec((R_PRE, D), lambda i: (i, 0)),
            pl.BlockSpec((D, D), lambda i: (0, 0)),
            pl.BlockSpec((1, D), lambda i: (0, 0)),
            pl.BlockSpec((1, D), lambda i: (0, 0)),
            pl.BlockSpec(memory_space=pltpu.SMEM),
            pl.BlockSpec((1, D), lambda i: (0, 0)),
            pl.BlockSpec(memory_space=pltpu.SMEM),
        ],
        out_specs=[pl.BlockSpec((R_PRE, NP * DP), lambda i: (i, 0))] + [
            pl.BlockSpec((R_PRE, 1), lambda i: (i, 0)),
            pl.BlockSpec((R_PRE, 1), lambda i: (i, 0)),
            pl.BlockSpec((16,), lambda i: (0,)),
        ],
        out_shape=[jax.ShapeDtypeStruct((N, NP * DP), jnp.float32)] + [
            jax.ShapeDtypeStruct((N, 1), jnp.float32),
            jax.ShapeDtypeStruct((N, 1), jnp.float32),
            jax.ShapeDtypeStruct((16,), jnp.float32),
        ],
        scratch_shapes=[pltpu.SMEM((2,), jnp.float32)],
    )(x, wt, b2d, w1, b1, w2, b2)


def _sc_body(src_hbm, dst_hbm, a1_hbm, a2_hbm, c_hbm, hp_hbm, out_hbm,
             a1_v, a2_v, c_v, si_v, di_v, gi_v, ev_v, rows_v, acc, gsem,
             ssem):
    cid = lax.axis_index("c")
    sid = lax.axis_index("s")
    w = cid * NS + sid
    rowbase = w * (EW // SUBCH)
    zbase = sid * ZROWS

    pltpu.sync_copy(a1_hbm, a1_v)
    pltpu.sync_copy(a2_hbm, a2_v)
    pltpu.sync_copy(c_hbm, c_v)
    pltpu.sync_copy(src_hbm.at[pl.ds(rowbase, EW // SUBCH)], si_v)
    pltpu.sync_copy(dst_hbm.at[pl.ds(rowbase, EW // SUBCH)], di_v)
    cvec = c_v[...]

    @pl.loop(0, EW // SUBCH)
    def _(r):
        @plsc.parallel_loop(0, SUBCH, 16, unroll=2)
        def _(k):
            s16 = si_v[r, pl.ds(k, 16)]
            d16 = di_v[r, pl.ds(k, 16)]
            v = (plsc.load_gather(a1_v, [s16])
                 + plsc.load_gather(a2_v, [d16]))
            v = jnp.where(v > 0, v, 0.01 * v)
            ev_v[pl.ds(r * SUBCH + k, 16)] = jnp.exp(v - cvec)

    def build_gi(ci, par, p):
        for j in range(KSUB):
            @plsc.parallel_loop(0, SUBCH, 16, unroll=2)
            def _(k):
                d16 = di_v[ci * KSUB + j, pl.ds(k, 16)]
                gi_v[par * KSUB + j, pl.ds(k, 16)] = d16 * NP + p

    def fire_gathers(par):
        for j in range(KSUB):
            pltpu.async_copy(hp_hbm.at[gi_v.at[par * KSUB + j]],
                             rows_v.at[pl.ds(par * CH + j * SUBCH, SUBCH)],
                             gsem)

    def wait_gathers(par):
        for j in range(KSUB):
            pltpu.make_async_copy(
                hp_hbm.at[gi_v.at[par * KSUB + j]],
                rows_v.at[pl.ds(par * CH + j * SUBCH, SUBCH)], gsem).wait()

    def fire_scatters(ci, par):
        for j in range(KSUB):
            pltpu.async_copy(rows_v.at[pl.ds(par * CH + j * SUBCH, SUBCH)],
                             acc.at[si_v.at[ci * KSUB + j]], ssem, add=True)

    def wait_scatters(ci, par):
        for j in range(KSUB):
            pltpu.make_async_copy(
                rows_v.at[pl.ds(par * CH + j * SUBCH, SUBCH)],
                acc.at[si_v.at[ci * KSUB + j]], ssem).wait()

    @pl.loop(0, NP)
    def _(p):
        @pl.loop(0, ZROWS)
        def _(r):
            for m in range(DP // 16):
                rows_v[r, pl.ds(m * 16, 16)] = jnp.zeros((16,), jnp.float32)
        pltpu.sync_copy(rows_v.at[pl.ds(0, ZROWS)],
                        acc.at[pl.ds(zbase, ZROWS)])
        plsc.subcore_barrier()

        @pl.loop(0, NCHUNK + 1)
        def _(ci):
            par = ci % 2
            nxt = 1 - par

            @pl.when(ci >= 2)
            def _():
                wait_scatters(ci - 2, par)

            @pl.when(ci < NCHUNK)
            def _():
                build_gi(ci, par, p)
                fire_gathers(par)

            @pl.when(ci >= 1)
            def _():
                wait_gathers(nxt)
                ebase = (ci - 1) * CH
                rbase = nxt * CH

                @plsc.parallel_loop(0, CH, 1, unroll=8)
                def _(e):
                    evb = plsc.load_gather(ev_v,
                                           [lax.broadcast(ebase + e, (16,))])
                    for m in range(DP // 16):
                        rows_v[rbase + e, pl.ds(m * 16, 16)] = (
                            rows_v[rbase + e, pl.ds(m * 16, 16)] * evb)

                fire_scatters(ci - 1, nxt)

        wait_scatters(NCHUNK - 1, (NCHUNK - 1) % 2)

        plsc.subcore_barrier()
        pltpu.sync_copy(acc.at[pl.ds(zbase, ZROWS)],
                        out_hbm.at[cid, p, pl.ds(zbase, ZROWS)])


def _sc_edges(src, dst, a1p, a2p, c, hps):
    mesh = plsc.VectorSubcoreMesh(core_axis_name="c", subcore_axis_name="s",
                                  num_cores=NC, num_subcores=NS)
    cp = pltpu.CompilerParams()
    if "needs_layout_passes" in pltpu.CompilerParams.__dataclass_fields__:
        cp = dataclasses.replace(cp, needs_layout_passes=False)
    if "use_tc_tiling_on_sc" in pltpu.CompilerParams.__dataclass_fields__:
        cp = dataclasses.replace(cp, use_tc_tiling_on_sc=False)
    kfn = pl.kernel(
        _sc_body,
        out_type=jax.ShapeDtypeStruct((NC, NP, NROWS, DP), jnp.float32),
        mesh=mesh,
        scratch_types=[
            pltpu.VMEM((NACC,), jnp.float32),
            pltpu.VMEM((NACC,), jnp.float32),
            pltpu.VMEM((16,), jnp.float32),
            pltpu.VMEM((EW // SUBCH, SUBCH), jnp.int32),
            pltpu.VMEM((EW // SUBCH, SUBCH), jnp.int32),
            pltpu.VMEM((2 * KSUB, SUBCH), jnp.int32),
            pltpu.VMEM((EW,), jnp.float32),
            pltpu.VMEM((2 * CH, DP), jnp.float32),
            pltpu.VMEM_SHARED((NROWS, DP), jnp.float32),
            pltpu.SemaphoreType.DMA,
            pltpu.SemaphoreType.DMA,
        ],
        compiler_params=cp,
    )
    return kfn(src, dst, a1p, a2p, c, hps)


def _post_body(p_ref, o_ref):
    s0 = p_ref[0, 0] + p_ref[1, 0]
    den = s0[:, DP - 1:DP]
    slabs = [s0[:, :FP]]
    for p in range(1, NP):
        sp = p_ref[0, p] + p_ref[1, p]
        slabs.append(sp[:, :FP] if p < NP - 1 else sp[:, :D - 4 * FP])
    num = jnp.concatenate(slabs, axis=1)
    o_ref[...] = jnp.where(den > 0, num / den, 0.0)


def _tc_post(p):
    return pl.pallas_call(
        _post_body,
        grid=(N // R_PRE,),
        in_specs=[pl.BlockSpec((NC, NP, R_PRE, DP), lambda i: (0, 0, i, 0))],
        out_specs=pl.BlockSpec((R_PRE, D), lambda i: (i, 0)),
        out_shape=jax.ShapeDtypeStruct((N, D), jnp.float32),
    )(p)


def kernel(features, indices, W, b, W_a1, b_a1, W_a2, b_a2):
    wt = W.T
    b2d = b.reshape(1, D)
    b1 = b_a1
    b2 = b_a2
    hp, a1, a2, c = _tc_pre(features, wt, b2d, W_a1, b1, W_a2, b2)
    hpq = hp.reshape(NP * N, DP)

    a1p = jnp.concatenate(
        [a1.reshape(N), jnp.full((NACC - N,), -jnp.inf, jnp.float32)])
    a2p = jnp.concatenate([a2.reshape(N), jnp.zeros((NACC - N,), jnp.float32)])

    idx = indices.astype(jnp.int32)
    pad = TOT - E
    pad_src = N + jnp.arange(pad, dtype=jnp.int32) % (NROWS - N)
    pad_dst = jnp.arange(pad, dtype=jnp.int32) % N
    src = jnp.concatenate([idx[0], pad_src])
    dst = jnp.concatenate([idx[1], pad_dst])
    src = src.reshape(TOT // SUBCH, SUBCH)
    dst = dst.reshape(TOT // SUBCH, SUBCH)

    p = _sc_edges(src, dst, a1p, a2p, c, hpq)
    return _tc_post(p)


# speedup vs baseline: 1.0022x; 1.0022x over previous
import dataclasses

import jax
import jax.numpy as jnp
from jax import lax
from jax.experimental import pallas as pl
from jax.experimental.pallas import tpu as pltpu
from jax.experimental.pallas import tpu_sc as plsc

N = 10000
D = 128
E = 320000
NACC = 10240
NROWS = 10240

NP = 5
FP = 31
DP = 32

NC = 2
NS = 16
NW = NC * NS
SUBCH = 128
CH = 512
KSUB = CH // SUBCH
EW = 10240
TOT = EW * NW
NCHUNK = EW // CH
ZROWS = NROWS // NS

R_PRE = 1000


def _pre_body(x_ref, wt_ref, b_ref, w1_ref, b1_ref, w2_ref, b2_ref,
              hp_ref, a1_ref, a2_ref, c_ref, mx_ref):
    i = pl.program_id(0)
    h = jnp.dot(x_ref[...], wt_ref[...], preferred_element_type=jnp.float32)
    h = h + b_ref[...]
    jc = lax.broadcasted_iota(jnp.int32, (D, NP * DP), 1)
    ir = lax.broadcasted_iota(jnp.int32, (D, NP * DP), 0)
    pp = jc // DP
    rr = jc % DP
    num_mask = (jc < 4 * DP) & (rr < FP) & (ir == pp * FP + rr)
    last_mask = (jc >= 4 * DP) & (jc < 4 * DP + 4) & (ir == 4 * FP + jc - 4 * DP)
    sel = jnp.where(num_mask | last_mask, 1.0, 0.0).astype(jnp.float32)
    hpv = lax.dot_general(h, sel, (((1,), (0,)), ((), ())),
                          precision=lax.Precision.HIGHEST,
                          preferred_element_type=jnp.float32)
    jc2 = lax.broadcasted_iota(jnp.int32, (R_PRE, NP * DP), 1)
    onescols = jnp.where(jc2 % DP == DP - 1, 1.0, 0.0).astype(jnp.float32)
    hp_ref[...] = hpv + onescols
    a1 = jnp.sum(h * w1_ref[...], axis=1, keepdims=True) + b1_ref[0]
    a2 = jnp.sum(h * w2_ref[...], axis=1, keepdims=True) + b2_ref[0]
    a1_ref[...] = a1
    a2_ref[...] = a2
    m1 = jnp.max(a1)
    m2 = jnp.max(a2)

    @pl.when(i == 0)
    def _():
        mx_ref[0] = m1
        mx_ref[1] = m2

    @pl.when(i > 0)
    def _():
        mx_ref[0] = jnp.maximum(mx_ref[0], m1)
        mx_ref[1] = jnp.maximum(mx_ref[1], m2)

    c_ref[...] = jnp.full((16,), jnp.maximum(mx_ref[0] + mx_ref[1], 0.0),
                          jnp.float32)


def _tc_pre(x, wt, b2d, w1, b1, w2, b2):
    return pl.pallas_call(
        _pre_body,
        grid=(N // R_PRE,),
        in_specs=[
            pl.BlockSpec((R_PRE, D), lambda i: (i, 0)),
            pl.BlockSpec((D, D), lambda i: (0, 0)),
            pl.BlockSpec((1, D), lambda i: (0, 0)),
            pl.BlockSpec((1, D), lambda i: (0, 0)),
            pl.BlockSpec(memory_space=pltpu.SMEM),
            pl.BlockSpec((1, D), lambda i: (0, 0)),
            pl.BlockSpec(memory_space=pltpu.SMEM),
        ],
        out_specs=[pl.BlockSpec((R_PRE, NP * DP), lambda i: (i, 0))] + [
            pl.BlockSpec((R_PRE, 1), lambda i: (i, 0)),
            pl.BlockSpec((R_PRE, 1), lambda i: (i, 0)),
            pl.BlockSpec((16,), lambda i: (0,)),
        ],
        out_shape=[jax.ShapeDtypeStruct((N, NP * DP), jnp.float32)] + [
            jax.ShapeDtypeStruct((N, 1), jnp.float32),
            jax.ShapeDtypeStruct((N, 1), jnp.float32),
            jax.ShapeDtypeStruct((16,), jnp.float32),
        ],
        scratch_shapes=[pltpu.SMEM((2,), jnp.float32)],
    )(x, wt, b2d, w1, b1, w2, b2)


def _sc_body(src_hbm, dst_hbm, a1_hbm, a2_hbm, c_hbm, hp_hbm, out_hbm,
             a1_v, a2_v, c_v, si_v, di_v, gi_v, ev_v, rows_v, acc, gsem,
             ssem):
    cid = lax.axis_index("c")
    sid = lax.axis_index("s")
    w = cid * NS + sid
    rowbase = w * (EW // SUBCH)
    zbase = sid * ZROWS

    pltpu.sync_copy(a1_hbm, a1_v)
    pltpu.sync_copy(a2_hbm, a2_v)
    pltpu.sync_copy(c_hbm, c_v)
    pltpu.sync_copy(src_hbm.at[pl.ds(rowbase, EW // SUBCH)], si_v)
    pltpu.sync_copy(dst_hbm.at[pl.ds(rowbase, EW // SUBCH)], di_v)
    cvec = c_v[...]

    @pl.loop(0, EW // SUBCH)
    def _(r):
        @plsc.parallel_loop(0, SUBCH, 16, unroll=2)
        def _(k):
            s16 = si_v[r, pl.ds(k, 16)]
            d16 = di_v[r, pl.ds(k, 16)]
            v = (plsc.load_gather(a1_v, [s16])
                 + plsc.load_gather(a2_v, [d16]))
            v = jnp.where(v > 0, v, 0.01 * v)
            ev_v[pl.ds(r * SUBCH + k, 16)] = jnp.exp(v - cvec)

    def build_gi(ci, par, p):
        for j in range(KSUB):
            @plsc.parallel_loop(0, SUBCH, 16, unroll=2)
            def _(k):
                d16 = di_v[ci * KSUB + j, pl.ds(k, 16)]
                gi_v[par * KSUB + j, pl.ds(k, 16)] = d16 * NP + p

    def fire_gathers(par):
        for j in range(KSUB):
            pltpu.async_copy(hp_hbm.at[gi_v.at[par * KSUB + j]],
                             rows_v.at[pl.ds(par * CH + j * SUBCH, SUBCH)],
                             gsem)

    def wait_gathers(par):
        for j in range(KSUB):
            pltpu.make_async_copy(
                hp_hbm.at[gi_v.at[par * KSUB + j]],
                rows_v.at[pl.ds(par * CH + j * SUBCH, SUBCH)], gsem).wait()

    def fire_scatters(ci, par):
        for j in range(KSUB):
            pltpu.async_copy(rows_v.at[pl.ds(par * CH + j * SUBCH, SUBCH)],
                             acc.at[si_v.at[ci * KSUB + j]], ssem, add=True)

    def wait_scatters(ci, par):
        for j in range(KSUB):
            pltpu.make_async_copy(
                rows_v.at[pl.ds(par * CH + j * SUBCH, SUBCH)],
                acc.at[si_v.at[ci * KSUB + j]], ssem).wait()

    @pl.loop(0, NP)
    def _(p):
        @pl.loop(0, ZROWS)
        def _(r):
            for m in range(DP // 16):
                rows_v[r, pl.ds(m * 16, 16)] = jnp.zeros((16,), jnp.float32)
        pltpu.sync_copy(rows_v.at[pl.ds(0, ZROWS)],
                        acc.at[pl.ds(zbase, ZROWS)])
        plsc.subcore_barrier()

        @pl.loop(0, NCHUNK + 1)
        def _(ci):
            par = ci % 2
            nxt = 1 - par

            @pl.when(ci >= 2)
            def _():
                wait_scatters(ci - 2, par)

            @pl.when(ci < NCHUNK)
            def _():
                build_gi(ci, par, p)
                fire_gathers(par)

            @pl.when(ci >= 1)
            def _():
                wait_gathers(nxt)
                ebase = (ci - 1) * CH
                rbase = nxt * CH

                @plsc.parallel_loop(0, CH, 1, unroll=4)
                def _(e):
                    evb = plsc.load_gather(ev_v,
                                           [lax.broadcast(ebase + e, (16,))])
                    for m in range(DP // 16):
                        rows_v[rbase + e, pl.ds(m * 16, 16)] = (
                            rows_v[rbase + e, pl.ds(m * 16, 16)] * evb)

                fire_scatters(ci - 1, nxt)

        wait_scatters(NCHUNK - 1, (NCHUNK - 1) % 2)

        plsc.subcore_barrier()
        pltpu.sync_copy(acc.at[pl.ds(zbase, ZROWS)],
                        out_hbm.at[cid, p, pl.ds(zbase, ZROWS)])


def _sc_edges(src, dst, a1p, a2p, c, hps):
    mesh = plsc.VectorSubcoreMesh(core_axis_name="c", subcore_axis_name="s",
                                  num_cores=NC, num_subcores=NS)
    cp = pltpu.CompilerParams()
    if "needs_layout_passes" in pltpu.CompilerParams.__dataclass_fields__:
        cp = dataclasses.replace(cp, needs_layout_passes=False)
    if "use_tc_tiling_on_sc" in pltpu.CompilerParams.__dataclass_fields__:
        cp = dataclasses.replace(cp, use_tc_tiling_on_sc=False)
    kfn = pl.kernel(
        _sc_body,
        out_type=jax.ShapeDtypeStruct((NC, NP, NROWS, DP), jnp.float32),
        mesh=mesh,
        scratch_types=[
            pltpu.VMEM((NACC,), jnp.float32),
            pltpu.VMEM((NACC,), jnp.float32),
            pltpu.VMEM((16,), jnp.float32),
            pltpu.VMEM((EW // SUBCH, SUBCH), jnp.int32),
            pltpu.VMEM((EW // SUBCH, SUBCH), jnp.int32),
            pltpu.VMEM((2 * KSUB, SUBCH), jnp.int32),
            pltpu.VMEM((EW,), jnp.float32),
            pltpu.VMEM((2 * CH, DP), jnp.float32),
            pltpu.VMEM_SHARED((NROWS, DP), jnp.float32),
            pltpu.SemaphoreType.DMA,
            pltpu.SemaphoreType.DMA,
        ],
        compiler_params=cp,
    )
    return kfn(src, dst, a1p, a2p, c, hps)


def _post_body(p_ref, o_ref):
    s0 = p_ref[0, 0] + p_ref[1, 0]
    den = s0[:, DP - 1:DP]
    slabs = [s0[:, :FP]]
    for p in range(1, NP):
        sp = p_ref[0, p] + p_ref[1, p]
        slabs.append(sp[:, :FP] if p < NP - 1 else sp[:, :D - 4 * FP])
    num = jnp.concatenate(slabs, axis=1)
    o_ref[...] = jnp.where(den > 0, num / den, 0.0)


def _tc_post(p):
    return pl.pallas_call(
        _post_body,
        grid=(N // R_PRE,),
        in_specs=[pl.BlockSpec((NC, NP, R_PRE, DP), lambda i: (0, 0, i, 0))],
        out_specs=pl.BlockSpec((R_PRE, D), lambda i: (i, 0)),
        out_shape=jax.ShapeDtypeStruct((N, D), jnp.float32),
    )(p)


def kernel(features, indices, W, b, W_a1, b_a1, W_a2, b_a2):
    wt = W.T
    b2d = b.reshape(1, D)
    b1 = b_a1
    b2 = b_a2
    hp, a1, a2, c = _tc_pre(features, wt, b2d, W_a1, b1, W_a2, b2)
    hpq = hp.reshape(NP * N, DP)

    a1p = jnp.concatenate(
        [a1.reshape(N), jnp.full((NACC - N,), -jnp.inf, jnp.float32)])
    a2p = jnp.concatenate([a2.reshape(N), jnp.zeros((NACC - N,), jnp.float32)])

    idx = indices.astype(jnp.int32)
    pad = TOT - E
    pad_src = N + jnp.arange(pad, dtype=jnp.int32) % (NROWS - N)
    pad_dst = jnp.arange(pad, dtype=jnp.int32) % N
    src = jnp.concatenate([idx[0], pad_src])
    dst = jnp.concatenate([idx[1], pad_dst])
    src = src.reshape(TOT // SUBCH, SUBCH)
    dst = dst.reshape(TOT // SUBCH, SUBCH)

    p = _sc_edges(src, dst, a1p, a2p, c, hpq)
    return _tc_post(p)
